# baseline (speedup 1.0000x reference)
import functools

import jax
import jax.numpy as jnp
from jax import lax
from jax.experimental import pallas as pl
from jax.experimental.pallas import tpu as pltpu
from jax.experimental.pallas import tpu_sc as plsc

_CHUNK = 128
_L = 16


def kernel(head_index, rel_type, tail_index, node_emb, rel_emb):
    B = head_index.shape[0]
    D = node_emb.shape[1]
    assert D == 64

    node2 = node_emb.reshape(-1, 2 * D)
    rel2 = rel_emb.reshape(-1, 2 * D)

    info = plsc.get_sparse_core_info()
    nw = info.num_cores * info.num_subcores
    b_per_w = B // nw
    assert B % nw == 0 and b_per_w % _CHUNK == 0
    n_chunks = b_per_w // _CHUNK
    n_groups = b_per_w // _L

    mesh = plsc.VectorSubcoreMesh(core_axis_name="c", subcore_axis_name="s")

    @functools.partial(
        pl.kernel,
        mesh=mesh,
        compiler_params=pltpu.CompilerParams(use_tc_tiling_on_sc=True),
        out_type=(
            jax.ShapeDtypeStruct((B // 2, 2 * D), jnp.float32),
            jax.ShapeDtypeStruct((B // 2, 2 * D), jnp.float32),
            jax.ShapeDtypeStruct((B // 2, 2 * D), jnp.float32),
        ),
        scratch_types=[
            pltpu.VMEM((b_per_w,), jnp.int32),
            pltpu.VMEM((b_per_w,), jnp.int32),
            pltpu.VMEM((b_per_w,), jnp.int32),
            pltpu.VMEM((b_per_w,), jnp.int32),
            pltpu.VMEM((b_per_w,), jnp.int32),
            pltpu.VMEM((b_per_w,), jnp.int32),
            pltpu.VMEM((b_per_w, 2 * D), jnp.float32),
            pltpu.VMEM((b_per_w // 2, 2 * D), jnp.float32),
            pltpu.SemaphoreType.DMA,
            pltpu.SemaphoreType.DMA,
        ],
    )
    def sc_gather(head_hbm, rel_hbm, tail_hbm, node_hbm, relemb_hbm,
                  head_out, rel_out, tail_out,
                  hidx, tidx, ridx, hq, tq, rq,
                  pairs, outbuf, sem_g, sem_w):
        cid = lax.axis_index("c")
        sid = lax.axis_index("s")
        wid = sid * info.num_cores + cid
        base = wid * b_per_w
        sl = pl.ds(base, b_per_w)

        pltpu.sync_copy(head_hbm.at[sl], hidx)
        pltpu.sync_copy(tail_hbm.at[sl], tidx)
        pltpu.sync_copy(rel_hbm.at[sl], ridx)

        def compute_q(g, _):
            s = pl.ds(g * _L, _L)
            hq[s] = lax.shift_right_logical(hidx[s], 1)
            tq[s] = lax.shift_right_logical(tidx[s], 1)
            rq[s] = lax.shift_right_logical(ridx[s], 1)
            return 0

        lax.fori_loop(0, n_groups, compute_q, 0)

        write_handles = []
        for tab_hbm, q, idx, out_hbm in (
            (node_hbm, hq, hidx, head_out),
            (node_hbm, tq, tidx, tail_out),
            (relemb_hbm, rq, ridx, rel_out),
        ):
            gathers = [
                pltpu.async_copy(
                    tab_hbm.at[q.at[pl.ds(c * _CHUNK, _CHUNK)]],
                    pairs.at[pl.ds(c * _CHUNK, _CHUNK)], sem_g)
                for c in range(n_chunks)
            ]
            for g in gathers:
                g.wait()
            if write_handles:
                write_handles.pop().wait()

            def select(g, _, idx=idx):
                iv = idx[pl.ds(g * _L, _L)]
                offv = lax.shift_left(
                    lax.bitwise_and(iv, jnp.int32(1)), 6)
                for jj in range(_L):
                    off = offv[jj]
                    src_row = g * _L + jj
                    out_row = g * (_L // 2) + (jj // 2)
                    col0 = (jj % 2) * D
                    for cc in range(D // _L):
                        outbuf[out_row, pl.ds(col0 + cc * _L, _L)] = (
                            pairs[src_row, pl.ds(off + cc * _L, _L)])
                return 0

            lax.fori_loop(0, n_groups, select, 0)
            write_handles.append(pltpu.async_copy(
                outbuf, out_hbm.at[pl.ds(wid * (b_per_w // 2),
                                         b_per_w // 2)], sem_w))
        write_handles.pop().wait()

    h2, r2, t2 = sc_gather(head_index, rel_type, tail_index, node2, rel2)
    return (h2.reshape(B, D), r2.reshape(B, D), t2.reshape(B, D))
